# initial kernel scaffold (unmeasured)
import jax
import jax.numpy as jnp
from jax import lax
from jax.experimental import pallas as pl
from jax.experimental.pallas import tpu as pltpu

N_DEV = 4
SQ = 2048
HQ = 8
DH = 128
D = HQ * DH
BLK = 64
QT = 512
SCALE = 0.08838834764831843
NEG = jnp.float32(-1e9)


def kernel(x, Wq, K_ext, V_ext, Wo):
    x2 = x.reshape(SQ, D)
    k2 = K_ext.reshape(SQ, D)
    v2 = V_ext.reshape(SQ, D)

    def body(x_ref, wq_ref, k_ref, v_ref, wo_ref, out_ref,
             q_scr, ctx_scr, m_scr, l_scr, ck, cv,
             ksend, krecv, vsend, vrecv):
        my = lax.axis_index("i")
        right = lax.rem(my + 1, N_DEV)

        def chunk_update(kc, vc, masked):
            for h in range(HQ):
                c0 = h * DH
                kh = kc[:, pl.ds(c0, DH)]
                vh = vc[:, pl.ds(c0, DH)]

                def qt_body(qi, _):
                    q0 = qi * QT
                    qh = q_scr[pl.ds(q0, QT), pl.ds(c0, DH)]
                    s = lax.dot_general(
                        qh, kh, (((1,), (1,)), ((), ())),
                        preferred_element_type=jnp.float32,
                    ) * SCALE
                    if masked:
                        ri = q0 + lax.broadcasted_iota(jnp.int32, (QT, SQ), 0)
                        ci = lax.broadcasted_iota(jnp.int32, (QT, SQ), 1)
                        s = jnp.where(ci // BLK <= ri // BLK, s, NEG)
                    m_old = m_scr[pl.ds(q0, QT), pl.ds(h, 1)]
                    l_old = l_scr[pl.ds(q0, QT), pl.ds(h, 1)]
                    m_new = jnp.maximum(m_old, jnp.max(s, axis=1, keepdims=True))
                    alpha = jnp.exp(m_old - m_new)
                    p = jnp.exp(s - m_new)
                    l_new = l_old * alpha + jnp.sum(p, axis=1, keepdims=True)
                    pv = jnp.dot(p, vh, preferred_element_type=jnp.float32)
                    ctx_scr[pl.ds(q0, QT), pl.ds(c0, DH)] = (
                        ctx_scr[pl.ds(q0, QT), pl.ds(c0, DH)] * alpha + pv
                    )
                    m_scr[pl.ds(q0, QT), pl.ds(h, 1)] = m_new
                    l_scr[pl.ds(q0, QT), pl.ds(h, 1)] = l_new
                    return 0

                lax.fori_loop(0, SQ // QT, qt_body, 0)

        def hop_rdma(hop):
            src_k = k_ref if hop == 0 else ck.at[hop - 1]
            src_v = v_ref if hop == 0 else cv.at[hop - 1]
            rk = pltpu.make_async_remote_copy(
                src_ref=src_k, dst_ref=ck.at[hop],
                send_sem=ksend.at[hop], recv_sem=krecv.at[hop],
                device_id=(right,), device_id_type=pl.DeviceIdType.MESH,
            )
            rv = pltpu.make_async_remote_copy(
                src_ref=src_v, dst_ref=cv.at[hop],
                send_sem=vsend.at[hop], recv_sem=vrecv.at[hop],
                device_id=(right,), device_id_type=pl.DeviceIdType.MESH,
            )
            rk.start()
            rv.start()
            return rk, rv

        rk, rv = hop_rdma(0)

        q_scr[...] = jnp.dot(x_ref[...], wq_ref[...],
                             preferred_element_type=jnp.float32)
        ctx_scr[...] = jnp.zeros_like(ctx_scr)
        m_scr[...] = jnp.full_like(m_scr, -1e30)
        l_scr[...] = jnp.zeros_like(l_scr)
        chunk_update(k_ref, v_ref, masked=True)

        for hop in range(N_DEV - 1):
            rk.wait()
            rv.wait()
            if hop + 1 < N_DEV - 1:
                rk, rv = hop_rdma(hop + 1)
            origin = lax.rem(my + (N_DEV - 1 - hop), N_DEV)

            @pl.when(origin < my)
            def _():
                chunk_update(ck.at[hop], cv.at[hop], masked=False)

        for h in range(HQ):
            c0 = h * DH
            ctx_scr[:, pl.ds(c0, DH)] = (
                ctx_scr[:, pl.ds(c0, DH)] / l_scr[:, pl.ds(h, 1)]
            )
        out_ref[...] = jnp.dot(ctx_scr[...], wo_ref[...],
                               preferred_element_type=jnp.float32)

    out = pl.pallas_call(
        body,
        out_shape=jax.ShapeDtypeStruct((SQ, D), jnp.float32),
        in_specs=[pl.BlockSpec(memory_space=pltpu.VMEM)] * 5,
        out_specs=pl.BlockSpec(memory_space=pltpu.VMEM),
        scratch_shapes=[
            pltpu.VMEM((SQ, D), jnp.float32),
            pltpu.VMEM((SQ, D), jnp.float32),
            pltpu.VMEM((SQ, HQ), jnp.float32),
            pltpu.VMEM((SQ, HQ), jnp.float32),
            pltpu.VMEM((N_DEV - 1, SQ, D), jnp.float32),
            pltpu.VMEM((N_DEV - 1, SQ, D), jnp.float32),
            pltpu.SemaphoreType.DMA((N_DEV - 1,)),
            pltpu.SemaphoreType.DMA((N_DEV - 1,)),
            pltpu.SemaphoreType.DMA((N_DEV - 1,)),
            pltpu.SemaphoreType.DMA((N_DEV - 1,)),
        ],
    )(x2, Wq, k2, v2, Wo)
    return out.reshape(1, SQ, D)


# baseline (device time: 665319 ns/iter reference)
import jax
import jax.numpy as jnp
from jax import lax
from jax.experimental import pallas as pl
from jax.experimental.pallas import tpu as pltpu

N_DEV = 4
SQ = 2048
HQ = 8
DH = 128
D = HQ * DH
BLK = 64
QT = 512
SCALE = 0.08838834764831843
NEG = -1e9


def kernel(x, Wq, K_ext, V_ext, Wo):
    x2 = x.reshape(SQ, D)
    k2 = K_ext.reshape(SQ, D)
    v2 = V_ext.reshape(SQ, D)

    def body(x_ref, wq_ref, k_ref, v_ref, wo_ref, out_ref, ck, cv,
             q_scr, ctx_scr, m_scr, l_scr,
             kh_t, vh_t, x_t,
             ksend, krecv, vsend, vrecv, csem):
        my = lax.axis_index("i")
        right = lax.rem(my + 1, N_DEV)

        def chunk_update(kc, vc, masked):
            for h in range(HQ):
                c0 = h * DH
                ckh = pltpu.make_async_copy(
                    kc.at[:, pl.ds(c0, DH)], kh_t, csem.at[0])
                cvh = pltpu.make_async_copy(
                    vc.at[:, pl.ds(c0, DH)], vh_t, csem.at[1])
                ckh.start()
                cvh.start()
                ckh.wait()
                cvh.wait()
                kh = kh_t[...]
                vh = vh_t[...]

                def qt_body(qi, _):
                    q0 = qi * QT
                    qh = q_scr[pl.ds(q0, QT), pl.ds(c0, DH)]
                    s = lax.dot_general(
                        qh, kh, (((1,), (1,)), ((), ())),
                        preferred_element_type=jnp.float32,
                    ) * SCALE
                    if masked:
                        rb = (q0 + lax.broadcasted_iota(jnp.int32, (QT, 1), 0)) // BLK
                        cb = lax.broadcasted_iota(jnp.int32, (1, SQ), 1) // BLK
                        s = jnp.where(cb <= rb, s, NEG)
                    m_old = m_scr[pl.ds(q0, QT), pl.ds(h, 1)]
                    l_old = l_scr[pl.ds(q0, QT), pl.ds(h, 1)]
                    m_new = jnp.maximum(m_old, jnp.max(s, axis=1, keepdims=True))
                    alpha = jnp.exp(m_old - m_new)
                    p = jnp.exp(s - m_new)
                    l_new = l_old * alpha + jnp.sum(p, axis=1, keepdims=True)
                    pv = jnp.dot(p, vh, preferred_element_type=jnp.float32)
                    ctx_scr[pl.ds(q0, QT), pl.ds(c0, DH)] = (
                        ctx_scr[pl.ds(q0, QT), pl.ds(c0, DH)] * alpha + pv
                    )
                    m_scr[pl.ds(q0, QT), pl.ds(h, 1)] = m_new
                    l_scr[pl.ds(q0, QT), pl.ds(h, 1)] = l_new
                    return 0

                lax.fori_loop(0, SQ // QT, qt_body, 0)

        def hop_rdma(hop):
            src_k = k_ref if hop == 0 else ck.at[hop - 1]
            src_v = v_ref if hop == 0 else cv.at[hop - 1]
            rk = pltpu.make_async_remote_copy(
                src_ref=src_k, dst_ref=ck.at[hop],
                send_sem=ksend.at[hop], recv_sem=krecv.at[hop],
                device_id=(right,), device_id_type=pl.DeviceIdType.MESH,
            )
            rv = pltpu.make_async_remote_copy(
                src_ref=src_v, dst_ref=cv.at[hop],
                send_sem=vsend.at[hop], recv_sem=vrecv.at[hop],
                device_id=(right,), device_id_type=pl.DeviceIdType.MESH,
            )
            rk.start()
            rv.start()
            return rk, rv

        rk, rv = hop_rdma(0)

        for qi in range(SQ // QT):
            cx = pltpu.make_async_copy(
                x_ref.at[pl.ds(qi * QT, QT), :], x_t, csem.at[0])
            cx.start()
            cx.wait()
            q_scr[pl.ds(qi * QT, QT), :] = jnp.dot(
                x_t[...], wq_ref[...], preferred_element_type=jnp.float32)
        ctx_scr[...] = jnp.zeros_like(ctx_scr)
        m_scr[...] = jnp.full_like(m_scr, -1e30)
        l_scr[...] = jnp.zeros_like(l_scr)
        chunk_update(k_ref, v_ref, masked=True)

        for hop in range(N_DEV - 1):
            rk.wait()
            rv.wait()
            if hop + 1 < N_DEV - 1:
                rk, rv = hop_rdma(hop + 1)
            origin = lax.rem(my + (N_DEV - 1 - hop), N_DEV)

            @pl.when(origin < my)
            def _():
                chunk_update(ck.at[hop], cv.at[hop], masked=False)

        for h in range(HQ):
            c0 = h * DH
            ctx_scr[:, pl.ds(c0, DH)] = (
                ctx_scr[:, pl.ds(c0, DH)] / l_scr[:, pl.ds(h, 1)]
            )
        out_ref[...] = jnp.dot(ctx_scr[...], wo_ref[...],
                               preferred_element_type=jnp.float32)

    out = pl.pallas_call(
        body,
        out_shape=[
            jax.ShapeDtypeStruct((SQ, D), jnp.float32),
            jax.ShapeDtypeStruct((N_DEV - 1, SQ, D), jnp.float32),
            jax.ShapeDtypeStruct((N_DEV - 1, SQ, D), jnp.float32),
        ],
        in_specs=[
            pl.BlockSpec(memory_space=pltpu.HBM),
            pl.BlockSpec(memory_space=pltpu.VMEM),
            pl.BlockSpec(memory_space=pltpu.HBM),
            pl.BlockSpec(memory_space=pltpu.HBM),
            pl.BlockSpec(memory_space=pltpu.VMEM),
        ],
        out_specs=[
            pl.BlockSpec(memory_space=pltpu.VMEM),
            pl.BlockSpec(memory_space=pltpu.HBM),
            pl.BlockSpec(memory_space=pltpu.HBM),
        ],
        scratch_shapes=[
            pltpu.VMEM((SQ, D), jnp.float32),
            pltpu.VMEM((SQ, D), jnp.float32),
            pltpu.VMEM((SQ, HQ), jnp.float32),
            pltpu.VMEM((SQ, HQ), jnp.float32),
            pltpu.VMEM((SQ, DH), jnp.float32),
            pltpu.VMEM((SQ, DH), jnp.float32),
            pltpu.VMEM((QT, D), jnp.float32),
            pltpu.SemaphoreType.DMA((N_DEV - 1,)),
            pltpu.SemaphoreType.DMA((N_DEV - 1,)),
            pltpu.SemaphoreType.DMA((N_DEV - 1,)),
            pltpu.SemaphoreType.DMA((N_DEV - 1,)),
            pltpu.SemaphoreType.DMA((2,)),
        ],
        compiler_params=pltpu.CompilerParams(
            vmem_limit_bytes=44 * 1024 * 1024,
        ),
    )(x2, Wq, k2, v2, Wo)
    return out[0].reshape(1, SQ, D)


# device time: 404001 ns/iter; 1.6468x vs baseline; 1.6468x over previous
import jax
import jax.numpy as jnp
from jax import lax
from jax.experimental import pallas as pl
from jax.experimental.pallas import tpu as pltpu

N_DEV = 4
SQ = 2048
HQ = 8
DH = 128
D = HQ * DH
BLK = 64
QT = 512
SCALE = 0.08838834764831843
NEG = -1e9


def kernel(x, Wq, K_ext, V_ext, Wo):
    bf16 = jnp.bfloat16
    x2 = x.reshape(SQ, D).astype(bf16)
    k2 = K_ext.reshape(SQ, D).astype(bf16)
    v2 = V_ext.reshape(SQ, D).astype(bf16)
    Wq = Wq.astype(bf16)
    Wo = Wo.astype(bf16)

    def body(x_ref, wq_ref, k_ref, v_ref, wo_ref, out_ref, ck, cv,
             q_scr, ctx_scr, m_scr, l_scr,
             kh_t, vh_t, x_t,
             ksend, krecv, vsend, vrecv, csem):
        my = lax.axis_index("i")
        right = lax.rem(my + 1, N_DEV)

        def chunk_update(kc, vc, masked):
            for h in range(HQ):
                c0 = h * DH
                ckh = pltpu.make_async_copy(
                    kc.at[:, pl.ds(c0, DH)], kh_t, csem.at[0])
                cvh = pltpu.make_async_copy(
                    vc.at[:, pl.ds(c0, DH)], vh_t, csem.at[1])
                ckh.start()
                cvh.start()
                ckh.wait()
                cvh.wait()
                kh = kh_t[...]
                vh = vh_t[...]

                def qt_body(qi, _):
                    q0 = qi * QT
                    qh = q_scr[pl.ds(q0, QT), pl.ds(c0, DH)]
                    s = lax.dot_general(
                        qh, kh, (((1,), (1,)), ((), ())),
                        preferred_element_type=jnp.float32,
                    ) * SCALE
                    if masked:
                        rb = (q0 + lax.broadcasted_iota(jnp.int32, (QT, 1), 0)) // BLK
                        cb = lax.broadcasted_iota(jnp.int32, (1, SQ), 1) // BLK
                        s = jnp.where(cb <= rb, s, NEG)
                    m_old = m_scr[pl.ds(q0, QT), pl.ds(h, 1)]
                    l_old = l_scr[pl.ds(q0, QT), pl.ds(h, 1)]
                    m_new = jnp.maximum(m_old, jnp.max(s, axis=1, keepdims=True))
                    alpha = jnp.exp(m_old - m_new)
                    p = jnp.exp(s - m_new)
                    l_new = l_old * alpha + jnp.sum(p, axis=1, keepdims=True)
                    pv = jnp.dot(p.astype(jnp.bfloat16), vh,
                                 preferred_element_type=jnp.float32)
                    ctx_scr[pl.ds(q0, QT), pl.ds(c0, DH)] = (
                        ctx_scr[pl.ds(q0, QT), pl.ds(c0, DH)] * alpha + pv
                    )
                    m_scr[pl.ds(q0, QT), pl.ds(h, 1)] = m_new
                    l_scr[pl.ds(q0, QT), pl.ds(h, 1)] = l_new
                    return 0

                lax.fori_loop(0, SQ // QT, qt_body, 0)

        def hop_rdma(hop):
            src_k = k_ref if hop == 0 else ck.at[hop - 1]
            src_v = v_ref if hop == 0 else cv.at[hop - 1]
            rk = pltpu.make_async_remote_copy(
                src_ref=src_k, dst_ref=ck.at[hop],
                send_sem=ksend.at[hop], recv_sem=krecv.at[hop],
                device_id=(right,), device_id_type=pl.DeviceIdType.MESH,
            )
            rv = pltpu.make_async_remote_copy(
                src_ref=src_v, dst_ref=cv.at[hop],
                send_sem=vsend.at[hop], recv_sem=vrecv.at[hop],
                device_id=(right,), device_id_type=pl.DeviceIdType.MESH,
            )
            rk.start()
            rv.start()
            return rk, rv

        rk, rv = hop_rdma(0)

        for qi in range(SQ // QT):
            cx = pltpu.make_async_copy(
                x_ref.at[pl.ds(qi * QT, QT), :], x_t, csem.at[0])
            cx.start()
            cx.wait()
            q_scr[pl.ds(qi * QT, QT), :] = jnp.dot(
                x_t[...], wq_ref[...], preferred_element_type=jnp.float32,
            ).astype(jnp.bfloat16)
        ctx_scr[...] = jnp.zeros_like(ctx_scr)
        m_scr[...] = jnp.full_like(m_scr, -1e30)
        l_scr[...] = jnp.zeros_like(l_scr)
        chunk_update(k_ref, v_ref, masked=True)

        for hop in range(N_DEV - 1):
            rk.wait()
            rv.wait()
            if hop + 1 < N_DEV - 1:
                rk, rv = hop_rdma(hop + 1)
            origin = lax.rem(my + (N_DEV - 1 - hop), N_DEV)

            @pl.when(origin < my)
            def _():
                chunk_update(ck.at[hop], cv.at[hop], masked=False)

        for h in range(HQ):
            c0 = h * DH
            ctx_scr[:, pl.ds(c0, DH)] = (
                ctx_scr[:, pl.ds(c0, DH)] / l_scr[:, pl.ds(h, 1)]
            )
        out_ref[...] = jnp.dot(ctx_scr[...].astype(jnp.bfloat16), wo_ref[...],
                               preferred_element_type=jnp.float32)

    out = pl.pallas_call(
        body,
        out_shape=[
            jax.ShapeDtypeStruct((SQ, D), jnp.float32),
            jax.ShapeDtypeStruct((N_DEV - 1, SQ, D), jnp.bfloat16),
            jax.ShapeDtypeStruct((N_DEV - 1, SQ, D), jnp.bfloat16),
        ],
        in_specs=[
            pl.BlockSpec(memory_space=pltpu.HBM),
            pl.BlockSpec(memory_space=pltpu.VMEM),
            pl.BlockSpec(memory_space=pltpu.HBM),
            pl.BlockSpec(memory_space=pltpu.HBM),
            pl.BlockSpec(memory_space=pltpu.VMEM),
        ],
        out_specs=[
            pl.BlockSpec(memory_space=pltpu.VMEM),
            pl.BlockSpec(memory_space=pltpu.HBM),
            pl.BlockSpec(memory_space=pltpu.HBM),
        ],
        scratch_shapes=[
            pltpu.VMEM((SQ, D), jnp.bfloat16),
            pltpu.VMEM((SQ, D), jnp.float32),
            pltpu.VMEM((SQ, HQ), jnp.float32),
            pltpu.VMEM((SQ, HQ), jnp.float32),
            pltpu.VMEM((SQ, DH), jnp.bfloat16),
            pltpu.VMEM((SQ, DH), jnp.bfloat16),
            pltpu.VMEM((QT, D), jnp.bfloat16),
            pltpu.SemaphoreType.DMA((N_DEV - 1,)),
            pltpu.SemaphoreType.DMA((N_DEV - 1,)),
            pltpu.SemaphoreType.DMA((N_DEV - 1,)),
            pltpu.SemaphoreType.DMA((N_DEV - 1,)),
            pltpu.SemaphoreType.DMA((2,)),
        ],
        compiler_params=pltpu.CompilerParams(
            vmem_limit_bytes=44 * 1024 * 1024,
        ),
    )(x2, Wq, k2, v2, Wo)
    return out[0].reshape(1, SQ, D)


# device time: 368451 ns/iter; 1.8057x vs baseline; 1.0965x over previous
import jax
import jax.numpy as jnp
from jax import lax
from jax.experimental import pallas as pl
from jax.experimental.pallas import tpu as pltpu

N_DEV = 4
SQ = 2048
HQ = 8
DH = 128
D = HQ * DH
BLK = 64
QT = 512
SCALE = 0.08838834764831843
NEG = -1e9

SLOT_A, SLOT_B, SLOT_C = 0, 1, 2


def kernel(x, Wq, K_ext, V_ext, Wo):
    bf16 = jnp.bfloat16
    x2 = x.reshape(SQ, D).astype(bf16)
    k2 = K_ext.reshape(SQ, D).astype(bf16)
    v2 = V_ext.reshape(SQ, D).astype(bf16)
    Wq = Wq.astype(bf16)
    Wo = Wo.astype(bf16)

    def body(x_ref, wq_ref, k_ref, v_ref, wo_ref, out_ref, ck, cv,
             q_scr, ctx_scr, m_scr, l_scr,
             kc_t, vc_t, x_t,
             ksend, krecv, vsend, vrecv, csem):
        my = lax.axis_index("i")
        right = lax.rem(my + 1, N_DEV)
        left = lax.rem(my + 3, N_DEV)

        def mk(src_k, src_v, slot, dev):
            rk = pltpu.make_async_remote_copy(
                src_ref=src_k, dst_ref=ck.at[slot],
                send_sem=ksend.at[slot], recv_sem=krecv.at[slot],
                device_id=(dev,), device_id_type=pl.DeviceIdType.MESH,
            )
            rv = pltpu.make_async_remote_copy(
                src_ref=src_v, dst_ref=cv.at[slot],
                send_sem=vsend.at[slot], recv_sem=vrecv.at[slot],
                device_id=(dev,), device_id_type=pl.DeviceIdType.MESH,
            )
            return rk, rv

        s1 = mk(k_ref, v_ref, SLOT_A, right)
        s3 = mk(ck.at[SLOT_A], cv.at[SLOT_A], SLOT_B, right)
        s2 = mk(k_ref, v_ref, SLOT_C, left)

        @pl.when(my < 3)
        def _():
            s1[0].start()
            s1[1].start()

        @pl.when(my == 0)
        def _():
            s2[0].start()
            s2[1].start()

        def chunk_update(kc, vc, masked):
            ckc = pltpu.make_async_copy(kc, kc_t, csem.at[0])
            cvc = pltpu.make_async_copy(vc, vc_t, csem.at[1])
            ckc.start()
            cvc.start()
            ckc.wait()
            cvc.wait()
            for h in range(HQ):
                c0 = h * DH
                kh = kc_t[:, pl.ds(c0, DH)]
                vh = vc_t[:, pl.ds(c0, DH)]

                def qt_body(qi, _):
                    q0 = qi * QT
                    qh = q_scr[pl.ds(q0, QT), pl.ds(c0, DH)]
                    s = lax.dot_general(
                        qh, kh, (((1,), (1,)), ((), ())),
                        preferred_element_type=jnp.float32,
                    ) * SCALE
                    if masked:
                        rb = (q0 + lax.broadcasted_iota(jnp.int32, (QT, 1), 0)) // BLK
                        cb = lax.broadcasted_iota(jnp.int32, (1, SQ), 1) // BLK
                        s = jnp.where(cb <= rb, s, NEG)
                    m_old = m_scr[pl.ds(q0, QT), pl.ds(h, 1)]
                    l_old = l_scr[pl.ds(q0, QT), pl.ds(h, 1)]
                    m_new = jnp.maximum(m_old, jnp.max(s, axis=1, keepdims=True))
                    alpha = jnp.exp(m_old - m_new)
                    p = jnp.exp(s - m_new)
                    l_new = l_old * alpha + jnp.sum(p, axis=1, keepdims=True)
                    pv = jnp.dot(p.astype(jnp.bfloat16), vh,
                                 preferred_element_type=jnp.float32)
                    ctx_scr[pl.ds(q0, QT), pl.ds(c0, DH)] = (
                        ctx_scr[pl.ds(q0, QT), pl.ds(c0, DH)] * alpha + pv
                    )
                    m_scr[pl.ds(q0, QT), pl.ds(h, 1)] = m_new
                    l_scr[pl.ds(q0, QT), pl.ds(h, 1)] = l_new
                    return 0

                lax.fori_loop(0, SQ // QT, qt_body, 0)

        for qi in range(SQ // QT):
            cx = pltpu.make_async_copy(
                x_ref.at[pl.ds(qi * QT, QT), :], x_t, csem.at[0])
            cx.start()
            cx.wait()
            q_scr[pl.ds(qi * QT, QT), :] = jnp.dot(
                x_t[...], wq_ref[...], preferred_element_type=jnp.float32,
            ).astype(jnp.bfloat16)
        ctx_scr[...] = jnp.zeros_like(ctx_scr)
        m_scr[...] = jnp.full_like(m_scr, -1e30)
        l_scr[...] = jnp.zeros_like(l_scr)
        chunk_update(k_ref, v_ref, masked=True)

        @pl.when(my >= 1)
        def _():
            s1[0].wait_recv()
            s1[1].wait_recv()

        @pl.when((my >= 1) & (my <= 2))
        def _():
            s3[0].start()
            s3[1].start()

        @pl.when(my >= 1)
        def _():
            chunk_update(ck.at[SLOT_A], cv.at[SLOT_A], masked=False)

        @pl.when(my == 3)
        def _():
            s2[0].wait_recv()
            s2[1].wait_recv()
            chunk_update(ck.at[SLOT_C], cv.at[SLOT_C], masked=False)

        @pl.when(my >= 2)
        def _():
            s3[0].wait_recv()
            s3[1].wait_recv()
            chunk_update(ck.at[SLOT_B], cv.at[SLOT_B], masked=False)

        @pl.when(my < 3)
        def _():
            s1[0].wait_send()
            s1[1].wait_send()

        @pl.when(my == 0)
        def _():
            s2[0].wait_send()
            s2[1].wait_send()

        @pl.when((my >= 1) & (my <= 2))
        def _():
            s3[0].wait_send()
            s3[1].wait_send()

        for h in range(HQ):
            c0 = h * DH
            ctx_scr[:, pl.ds(c0, DH)] = (
                ctx_scr[:, pl.ds(c0, DH)] / l_scr[:, pl.ds(h, 1)]
            )
        out_ref[...] = jnp.dot(ctx_scr[...].astype(jnp.bfloat16), wo_ref[...],
                               preferred_element_type=jnp.float32)

    out = pl.pallas_call(
        body,
        out_shape=[
            jax.ShapeDtypeStruct((SQ, D), jnp.float32),
            jax.ShapeDtypeStruct((3, SQ, D), jnp.bfloat16),
            jax.ShapeDtypeStruct((3, SQ, D), jnp.bfloat16),
        ],
        in_specs=[
            pl.BlockSpec(memory_space=pltpu.HBM),
            pl.BlockSpec(memory_space=pltpu.VMEM),
            pl.BlockSpec(memory_space=pltpu.HBM),
            pl.BlockSpec(memory_space=pltpu.HBM),
            pl.BlockSpec(memory_space=pltpu.VMEM),
        ],
        out_specs=[
            pl.BlockSpec(memory_space=pltpu.VMEM),
            pl.BlockSpec(memory_space=pltpu.HBM),
            pl.BlockSpec(memory_space=pltpu.HBM),
        ],
        scratch_shapes=[
            pltpu.VMEM((SQ, D), jnp.bfloat16),
            pltpu.VMEM((SQ, D), jnp.float32),
            pltpu.VMEM((SQ, HQ), jnp.float32),
            pltpu.VMEM((SQ, HQ), jnp.float32),
            pltpu.VMEM((SQ, D), jnp.bfloat16),
            pltpu.VMEM((SQ, D), jnp.bfloat16),
            pltpu.VMEM((QT, D), jnp.bfloat16),
            pltpu.SemaphoreType.DMA((3,)),
            pltpu.SemaphoreType.DMA((3,)),
            pltpu.SemaphoreType.DMA((3,)),
            pltpu.SemaphoreType.DMA((3,)),
            pltpu.SemaphoreType.DMA((2,)),
        ],
        compiler_params=pltpu.CompilerParams(
            vmem_limit_bytes=48 * 1024 * 1024,
        ),
    )(x2, Wq, k2, v2, Wo)
    return out[0].reshape(1, SQ, D)


# device time: 327331 ns/iter; 2.0326x vs baseline; 1.1256x over previous
import jax
import jax.numpy as jnp
from jax import lax
from jax.experimental import pallas as pl
from jax.experimental.pallas import tpu as pltpu

N_DEV = 4
SQ = 2048
HQ = 8
DH = 128
D = HQ * DH
BLK = 64
QT = 512
SCALE = 0.08838834764831843
NEG = -1e9

SLOT_A, SLOT_B = 0, 1
XQ, XP, XM = 0, 1, 2


def kernel(x, Wq, K_ext, V_ext, Wo):
    bf16 = jnp.bfloat16
    x2 = x.reshape(SQ, D).astype(bf16)
    k2 = K_ext.reshape(SQ, D).astype(bf16)
    v2 = V_ext.reshape(SQ, D).astype(bf16)
    Wq = Wq.astype(bf16)
    Wo = Wo.astype(bf16)

    def body(x_ref, wq_ref, k_ref, v_ref, wo_ref, out_ref, ck, cv,
             q_scr, ctx_scr, m_scr, l_scr,
             kc_t, vc_t, x_t, q3_t, ml3_t, o_t,
             ksend, krecv, vsend, vrecv, xsend, xrecv, csem):
        my = lax.axis_index("i")
        right = lax.rem(my + 1, N_DEV)
        left = lax.rem(my + 3, N_DEV)

        def mk_kv(src_k, src_v, slot, dev):
            rk = pltpu.make_async_remote_copy(
                src_ref=src_k, dst_ref=ck.at[slot],
                send_sem=ksend.at[slot], recv_sem=krecv.at[slot],
                device_id=(dev,), device_id_type=pl.DeviceIdType.MESH,
            )
            rv = pltpu.make_async_remote_copy(
                src_ref=src_v, dst_ref=cv.at[slot],
                send_sem=vsend.at[slot], recv_sem=vrecv.at[slot],
                device_id=(dev,), device_id_type=pl.DeviceIdType.MESH,
            )
            return rk, rv

        def mk_x(src, dst, idx, dev):
            return pltpu.make_async_remote_copy(
                src_ref=src, dst_ref=dst,
                send_sem=xsend.at[idx], recv_sem=xrecv.at[idx],
                device_id=(dev,), device_id_type=pl.DeviceIdType.MESH,
            )

        s1 = mk_kv(k_ref, v_ref, SLOT_A, right)
        s3 = mk_kv(ck.at[SLOT_A], cv.at[SLOT_A], SLOT_B, right)
        sQ = mk_x(q_scr, q3_t, XQ, right)
        sP = mk_x(q3_t, q3_t, XP, left)
        sM = mk_x(ml3_t, ml3_t, XM, left)

        @pl.when(my < 3)
        def _():
            s1[0].start()
            s1[1].start()

        def chunk_update(kc, vc, masked):
            ckc = pltpu.make_async_copy(kc, kc_t, csem.at[0])
            cvc = pltpu.make_async_copy(vc, vc_t, csem.at[1])
            ckc.start()
            cvc.start()
            ckc.wait()
            cvc.wait()
            for h in range(HQ):
                c0 = h * DH
                kh = kc_t[:, pl.ds(c0, DH)]
                vh = vc_t[:, pl.ds(c0, DH)]

                def qt_body(qi, _):
                    q0 = qi * QT
                    qh = q_scr[pl.ds(q0, QT), pl.ds(c0, DH)]
                    s = lax.dot_general(
                        qh, kh, (((1,), (1,)), ((), ())),
                        preferred_element_type=jnp.float32,
                    ) * SCALE
                    if masked:
                        rb = (q0 + lax.broadcasted_iota(jnp.int32, (QT, 1), 0)) // BLK
                        cb = lax.broadcasted_iota(jnp.int32, (1, SQ), 1) // BLK
                        s = jnp.where(cb <= rb, s, NEG)
                    m_old = m_scr[pl.ds(q0, QT), pl.ds(h, 1)]
                    l_old = l_scr[pl.ds(q0, QT), pl.ds(h, 1)]
                    m_new = jnp.maximum(m_old, jnp.max(s, axis=1, keepdims=True))
                    alpha = jnp.exp(m_old - m_new)
                    p = jnp.exp(s - m_new)
                    l_new = l_old * alpha + jnp.sum(p, axis=1, keepdims=True)
                    pv = jnp.dot(p.astype(jnp.bfloat16), vh,
                                 preferred_element_type=jnp.float32)
                    ctx_scr[pl.ds(q0, QT), pl.ds(c0, DH)] = (
                        ctx_scr[pl.ds(q0, QT), pl.ds(c0, DH)] * alpha + pv
                    )
                    m_scr[pl.ds(q0, QT), pl.ds(h, 1)] = m_new
                    l_scr[pl.ds(q0, QT), pl.ds(h, 1)] = l_new
                    return 0

                lax.fori_loop(0, SQ // QT, qt_body, 0)

        for qi in range(SQ // QT):
            cx = pltpu.make_async_copy(
                x_ref.at[pl.ds(qi * QT, QT), :], x_t, csem.at[0])
            cx.start()
            cx.wait()
            q_scr[pl.ds(qi * QT, QT), :] = jnp.dot(
                x_t[...], wq_ref[...], preferred_element_type=jnp.float32,
            ).astype(jnp.bfloat16)

        @pl.when(my == 3)
        def _():
            sQ.start()

        ctx_scr[...] = jnp.zeros_like(ctx_scr)
        m_scr[...] = jnp.full_like(m_scr, -1e30)
        l_scr[...] = jnp.zeros_like(l_scr)
        chunk_update(k_ref, v_ref, masked=True)

        @pl.when(my == 0)
        def _():
            sQ.wait_recv()
            for h in range(HQ):
                c0 = h * DH
                kh = kc_t[:, pl.ds(c0, DH)]
                vh = vc_t[:, pl.ds(c0, DH)]

                def qt_body(qi, _):
                    q0 = qi * QT
                    qh = q3_t[pl.ds(q0, QT), pl.ds(c0, DH)]
                    s = lax.dot_general(
                        qh, kh, (((1,), (1,)), ((), ())),
                        preferred_element_type=jnp.float32,
                    ) * SCALE
                    m3 = jnp.max(s, axis=1, keepdims=True)
                    p = jnp.exp(s - m3)
                    l3 = jnp.sum(p, axis=1, keepdims=True)
                    pv = jnp.dot(p.astype(jnp.bfloat16), vh,
                                 preferred_element_type=jnp.float32)
                    q3_t[pl.ds(q0, QT), pl.ds(c0, DH)] = pv.astype(jnp.bfloat16)
                    ml3_t[pl.ds(q0, QT), pl.ds(h, 1)] = m3
                    ml3_t[pl.ds(q0, QT), pl.ds(HQ + h, 1)] = l3
                    return 0

                lax.fori_loop(0, SQ // QT, qt_body, 0)
            sP.start()
            sM.start()

        @pl.when(my >= 1)
        def _():
            s1[0].wait_recv()
            s1[1].wait_recv()

        @pl.when((my >= 1) & (my <= 2))
        def _():
            s3[0].start()
            s3[1].start()

        @pl.when(my >= 1)
        def _():
            chunk_update(ck.at[SLOT_A], cv.at[SLOT_A], masked=False)

        @pl.when(my >= 2)
        def _():
            s3[0].wait_recv()
            s3[1].wait_recv()
            chunk_update(ck.at[SLOT_B], cv.at[SLOT_B], masked=False)

        @pl.when(my == 3)
        def _():
            sP.wait_recv()
            sM.wait_recv()
            for h in range(HQ):
                c0 = h * DH
                m_old = m_scr[:, pl.ds(h, 1)]
                l_old = l_scr[:, pl.ds(h, 1)]
                m3 = ml3_t[:, pl.ds(h, 1)]
                l3 = ml3_t[:, pl.ds(HQ + h, 1)]
                mm = jnp.maximum(m_old, m3)
                a = jnp.exp(m_old - mm)
                a3 = jnp.exp(m3 - mm)
                ctx_scr[:, pl.ds(c0, DH)] = (
                    ctx_scr[:, pl.ds(c0, DH)] * a
                    + q3_t[:, pl.ds(c0, DH)].astype(jnp.float32) * a3
                )
                l_scr[:, pl.ds(h, 1)] = l_old * a + l3 * a3

        @pl.when(my < 3)
        def _():
            s1[0].wait_send()
            s1[1].wait_send()

        @pl.when((my >= 1) & (my <= 2))
        def _():
            s3[0].wait_send()
            s3[1].wait_send()

        @pl.when(my == 3)
        def _():
            sQ.wait_send()

        @pl.when(my == 0)
        def _():
            sP.wait_send()
            sM.wait_send()

        for h in range(HQ):
            c0 = h * DH
            ctx_scr[:, pl.ds(c0, DH)] = (
                ctx_scr[:, pl.ds(c0, DH)] / l_scr[:, pl.ds(h, 1)]
            )
        for qi in range(SQ // QT):
            o_t[...] = jnp.dot(
                ctx_scr[pl.ds(qi * QT, QT), :].astype(jnp.bfloat16),
                wo_ref[...], preferred_element_type=jnp.float32)
            co = pltpu.make_async_copy(
                o_t, out_ref.at[pl.ds(qi * QT, QT), :], csem.at[0])
            co.start()
            co.wait()

    out = pl.pallas_call(
        body,
        out_shape=[
            jax.ShapeDtypeStruct((SQ, D), jnp.float32),
            jax.ShapeDtypeStruct((2, SQ, D), jnp.bfloat16),
            jax.ShapeDtypeStruct((2, SQ, D), jnp.bfloat16),
        ],
        in_specs=[
            pl.BlockSpec(memory_space=pltpu.HBM),
            pl.BlockSpec(memory_space=pltpu.VMEM),
            pl.BlockSpec(memory_space=pltpu.HBM),
            pl.BlockSpec(memory_space=pltpu.HBM),
            pl.BlockSpec(memory_space=pltpu.VMEM),
        ],
        out_specs=[
            pl.BlockSpec(memory_space=pltpu.HBM),
            pl.BlockSpec(memory_space=pltpu.HBM),
            pl.BlockSpec(memory_space=pltpu.HBM),
        ],
        scratch_shapes=[
            pltpu.VMEM((SQ, D), jnp.bfloat16),
            pltpu.VMEM((SQ, D), jnp.float32),
            pltpu.VMEM((SQ, HQ), jnp.float32),
            pltpu.VMEM((SQ, HQ), jnp.float32),
            pltpu.VMEM((SQ, D), jnp.bfloat16),
            pltpu.VMEM((SQ, D), jnp.bfloat16),
            pltpu.VMEM((QT, D), jnp.bfloat16),
            pltpu.VMEM((SQ, D), jnp.bfloat16),
            pltpu.VMEM((SQ, 2 * HQ), jnp.float32),
            pltpu.VMEM((QT, D), jnp.float32),
            pltpu.SemaphoreType.DMA((2,)),
            pltpu.SemaphoreType.DMA((2,)),
            pltpu.SemaphoreType.DMA((2,)),
            pltpu.SemaphoreType.DMA((2,)),
            pltpu.SemaphoreType.DMA((3,)),
            pltpu.SemaphoreType.DMA((3,)),
            pltpu.SemaphoreType.DMA((2,)),
        ],
        compiler_params=pltpu.CompilerParams(
            vmem_limit_bytes=48 * 1024 * 1024,
        ),
    )(x2, Wq, k2, v2, Wo)
    return out[0].reshape(1, SQ, D)


# device time: 312844 ns/iter; 2.1267x vs baseline; 1.0463x over previous
import jax
import jax.numpy as jnp
from jax import lax
from jax.experimental import pallas as pl
from jax.experimental.pallas import tpu as pltpu

N_DEV = 4
SQ = 2048
HQ = 8
DH = 128
D = HQ * DH
BLK = 64
QT = 512
SCALE = 0.08838834764831843
NEG = -1e9

SLOT_A, SLOT_B = 0, 1
XQ, XP, XM = 0, 1, 2


def kernel(x, Wq, K_ext, V_ext, Wo):
    bf16 = jnp.bfloat16
    x2 = x.reshape(SQ, D).astype(bf16)
    k2 = K_ext.reshape(SQ, D).astype(bf16)
    v2 = V_ext.reshape(SQ, D).astype(bf16)
    Wq = Wq.astype(bf16)
    Wo = Wo.astype(bf16)

    def body(x_ref, wq_ref, k_ref, v_ref, wo_ref, out_ref, ck, cv,
             q_scr, ctx_scr, m_scr, l_scr,
             kc_t, vc_t, x_t, q3_t, ml3_t, o_t,
             ksend, krecv, vsend, vrecv, xsend, xrecv, csem):
        my = lax.axis_index("i")
        right = lax.rem(my + 1, N_DEV)
        left = lax.rem(my + 3, N_DEV)

        def mk_kv(src_k, src_v, slot, dev):
            rk = pltpu.make_async_remote_copy(
                src_ref=src_k, dst_ref=ck.at[slot],
                send_sem=ksend.at[slot], recv_sem=krecv.at[slot],
                device_id=(dev,), device_id_type=pl.DeviceIdType.MESH,
            )
            rv = pltpu.make_async_remote_copy(
                src_ref=src_v, dst_ref=cv.at[slot],
                send_sem=vsend.at[slot], recv_sem=vrecv.at[slot],
                device_id=(dev,), device_id_type=pl.DeviceIdType.MESH,
            )
            return rk, rv

        def mk_x(src, dst, idx, dev):
            return pltpu.make_async_remote_copy(
                src_ref=src, dst_ref=dst,
                send_sem=xsend.at[idx], recv_sem=xrecv.at[idx],
                device_id=(dev,), device_id_type=pl.DeviceIdType.MESH,
            )

        s1 = mk_kv(k_ref, v_ref, SLOT_A, right)
        s3 = mk_kv(ck.at[SLOT_A], cv.at[SLOT_A], SLOT_B, right)
        sQ = mk_x(q_scr, q3_t, XQ, right)
        sP = mk_x(q3_t, q3_t, XP, left)
        sM = mk_x(ml3_t, ml3_t, XM, left)

        @pl.when(my < 3)
        def _():
            s1[0].start()
            s1[1].start()

        def chunk_update(kc, vc, masked):
            ckc = pltpu.make_async_copy(kc, kc_t, csem.at[0])
            cvc = pltpu.make_async_copy(vc, vc_t, csem.at[1])
            ckc.start()
            cvc.start()
            ckc.wait()
            cvc.wait()
            for h in range(HQ):
                c0 = h * DH

                def qt_step(qi, q0, kvl):
                    qh = q_scr[pl.ds(q0, QT), pl.ds(c0, DH)]
                    kh = kc_t[pl.ds(0, kvl), pl.ds(c0, DH)]
                    vh = vc_t[pl.ds(0, kvl), pl.ds(c0, DH)]
                    s = lax.dot_general(
                        qh, kh, (((1,), (1,)), ((), ())),
                        preferred_element_type=jnp.float32,
                    ) * SCALE
                    if masked:
                        rb = (q0 + lax.broadcasted_iota(jnp.int32, (QT, 1), 0)) // BLK
                        cb = lax.broadcasted_iota(jnp.int32, (1, kvl), 1) // BLK
                        s = jnp.where(cb <= rb, s, NEG)
                    m_old = m_scr[pl.ds(q0, QT), pl.ds(h, 1)]
                    l_old = l_scr[pl.ds(q0, QT), pl.ds(h, 1)]
                    m_new = jnp.maximum(m_old, jnp.max(s, axis=1, keepdims=True))
                    alpha = jnp.exp(m_old - m_new)
                    p = jnp.exp(s - m_new)
                    l_new = l_old * alpha + jnp.sum(p, axis=1, keepdims=True)
                    pv = jnp.dot(p.astype(jnp.bfloat16), vh,
                                 preferred_element_type=jnp.float32)
                    ctx_scr[pl.ds(q0, QT), pl.ds(c0, DH)] = (
                        ctx_scr[pl.ds(q0, QT), pl.ds(c0, DH)] * alpha + pv
                    )
                    m_scr[pl.ds(q0, QT), pl.ds(h, 1)] = m_new
                    l_scr[pl.ds(q0, QT), pl.ds(h, 1)] = l_new

                if masked:
                    for qi in range(SQ // QT):
                        qt_step(qi, qi * QT, (qi + 1) * QT)
                else:
                    def qt_body(qi, _):
                        qt_step(qi, qi * QT, SQ)
                        return 0

                    lax.fori_loop(0, SQ // QT, qt_body, 0)

        for qi in range(SQ // QT):
            cx = pltpu.make_async_copy(
                x_ref.at[pl.ds(qi * QT, QT), :], x_t, csem.at[0])
            cx.start()
            cx.wait()
            q_scr[pl.ds(qi * QT, QT), :] = jnp.dot(
                x_t[...], wq_ref[...], preferred_element_type=jnp.float32,
            ).astype(jnp.bfloat16)

        @pl.when(my == 3)
        def _():
            sQ.start()

        ctx_scr[...] = jnp.zeros_like(ctx_scr)
        m_scr[...] = jnp.full_like(m_scr, -1e30)
        l_scr[...] = jnp.zeros_like(l_scr)
        chunk_update(k_ref, v_ref, masked=True)

        @pl.when(my == 0)
        def _():
            sQ.wait_recv()
            for h in range(HQ):
                c0 = h * DH
                kh = kc_t[:, pl.ds(c0, DH)]
                vh = vc_t[:, pl.ds(c0, DH)]

                def qt_body(qi, _):
                    q0 = qi * QT
                    qh = q3_t[pl.ds(q0, QT), pl.ds(c0, DH)]
                    s = lax.dot_general(
                        qh, kh, (((1,), (1,)), ((), ())),
                        preferred_element_type=jnp.float32,
                    ) * SCALE
                    m3 = jnp.max(s, axis=1, keepdims=True)
                    p = jnp.exp(s - m3)
                    l3 = jnp.sum(p, axis=1, keepdims=True)
                    pv = jnp.dot(p.astype(jnp.bfloat16), vh,
                                 preferred_element_type=jnp.float32)
                    q3_t[pl.ds(q0, QT), pl.ds(c0, DH)] = pv.astype(jnp.bfloat16)
                    ml3_t[pl.ds(q0, QT), pl.ds(h, 1)] = m3
                    ml3_t[pl.ds(q0, QT), pl.ds(HQ + h, 1)] = l3
                    return 0

                lax.fori_loop(0, SQ // QT, qt_body, 0)
            sP.start()
            sM.start()

        @pl.when(my >= 1)
        def _():
            s1[0].wait_recv()
            s1[1].wait_recv()

        @pl.when((my >= 1) & (my <= 2))
        def _():
            s3[0].start()
            s3[1].start()

        @pl.when(my >= 1)
        def _():
            chunk_update(ck.at[SLOT_A], cv.at[SLOT_A], masked=False)

        @pl.when(my >= 2)
        def _():
            s3[0].wait_recv()
            s3[1].wait_recv()
            chunk_update(ck.at[SLOT_B], cv.at[SLOT_B], masked=False)

        @pl.when(my == 3)
        def _():
            sP.wait_recv()
            sM.wait_recv()
            for h in range(HQ):
                c0 = h * DH
                m_old = m_scr[:, pl.ds(h, 1)]
                l_old = l_scr[:, pl.ds(h, 1)]
                m3 = ml3_t[:, pl.ds(h, 1)]
                l3 = ml3_t[:, pl.ds(HQ + h, 1)]
                mm = jnp.maximum(m_old, m3)
                a = jnp.exp(m_old - mm)
                a3 = jnp.exp(m3 - mm)
                ctx_scr[:, pl.ds(c0, DH)] = (
                    ctx_scr[:, pl.ds(c0, DH)] * a
                    + q3_t[:, pl.ds(c0, DH)].astype(jnp.float32) * a3
                )
                l_scr[:, pl.ds(h, 1)] = l_old * a + l3 * a3

        @pl.when(my < 3)
        def _():
            s1[0].wait_send()
            s1[1].wait_send()

        @pl.when((my >= 1) & (my <= 2))
        def _():
            s3[0].wait_send()
            s3[1].wait_send()

        @pl.when(my == 3)
        def _():
            sQ.wait_send()

        @pl.when(my == 0)
        def _():
            sP.wait_send()
            sM.wait_send()

        for h in range(HQ):
            c0 = h * DH
            ctx_scr[:, pl.ds(c0, DH)] = (
                ctx_scr[:, pl.ds(c0, DH)] / l_scr[:, pl.ds(h, 1)]
            )
        for qi in range(SQ // QT):
            o_t[...] = jnp.dot(
                ctx_scr[pl.ds(qi * QT, QT), :].astype(jnp.bfloat16),
                wo_ref[...], preferred_element_type=jnp.float32)
            co = pltpu.make_async_copy(
                o_t, out_ref.at[pl.ds(qi * QT, QT), :], csem.at[0])
            co.start()
            co.wait()

    out = pl.pallas_call(
        body,
        out_shape=[
            jax.ShapeDtypeStruct((SQ, D), jnp.float32),
            jax.ShapeDtypeStruct((2, SQ, D), jnp.bfloat16),
            jax.ShapeDtypeStruct((2, SQ, D), jnp.bfloat16),
        ],
        in_specs=[
            pl.BlockSpec(memory_space=pltpu.HBM),
            pl.BlockSpec(memory_space=pltpu.VMEM),
            pl.BlockSpec(memory_space=pltpu.HBM),
            pl.BlockSpec(memory_space=pltpu.HBM),
            pl.BlockSpec(memory_space=pltpu.VMEM),
        ],
        out_specs=[
            pl.BlockSpec(memory_space=pltpu.HBM),
            pl.BlockSpec(memory_space=pltpu.HBM),
            pl.BlockSpec(memory_space=pltpu.HBM),
        ],
        scratch_shapes=[
            pltpu.VMEM((SQ, D), jnp.bfloat16),
            pltpu.VMEM((SQ, D), jnp.float32),
            pltpu.VMEM((SQ, HQ), jnp.float32),
            pltpu.VMEM((SQ, HQ), jnp.float32),
            pltpu.VMEM((SQ, D), jnp.bfloat16),
            pltpu.VMEM((SQ, D), jnp.bfloat16),
            pltpu.VMEM((QT, D), jnp.bfloat16),
            pltpu.VMEM((SQ, D), jnp.bfloat16),
            pltpu.VMEM((SQ, 2 * HQ), jnp.float32),
            pltpu.VMEM((QT, D), jnp.float32),
            pltpu.SemaphoreType.DMA((2,)),
            pltpu.SemaphoreType.DMA((2,)),
            pltpu.SemaphoreType.DMA((2,)),
            pltpu.SemaphoreType.DMA((2,)),
            pltpu.SemaphoreType.DMA((3,)),
            pltpu.SemaphoreType.DMA((3,)),
            pltpu.SemaphoreType.DMA((2,)),
        ],
        compiler_params=pltpu.CompilerParams(
            vmem_limit_bytes=48 * 1024 * 1024,
        ),
    )(x2, Wq, k2, v2, Wo)
    return out[0].reshape(1, SQ, D)
